# Initial kernel scaffold; baseline (speedup 1.0000x reference)
import jax
import jax.numpy as jnp
from jax.experimental import pallas as pl


def kernel(x, edge_index, W1, att_src1, att_dst1, b1, W2, att_src2, att_dst2, b2):
    raise NotImplementedError("write your pallas kernel here")


# speedup vs baseline: 1.0293x; 1.0293x over previous
import jax
import jax.numpy as jnp
from jax.experimental import pallas as pl

N = 10000
HEADS = 8
HID = 16


def _mm_body(x_ref, w_ref, o_ref):
    o_ref[...] = jnp.dot(x_ref[...], w_ref[...], preferred_element_type=jnp.float32)


def _matmul(x, w):
    M, K = x.shape
    _, C = w.shape
    BM = 1000
    return pl.pallas_call(
        _mm_body,
        grid=(M // BM,),
        in_specs=[
            pl.BlockSpec((BM, K), lambda i: (i, 0)),
            pl.BlockSpec((K, C), lambda i: (0, 0)),
        ],
        out_specs=pl.BlockSpec((BM, C), lambda i: (i, 0)),
        out_shape=jax.ShapeDtypeStruct((M, C), jnp.float32),
    )(x, w)


def _gat_conv(x, s, d, W, att_src, att_dst, bias, heads, out_ch, slope=0.2):
    n = x.shape[0]
    h = _matmul(x, W).reshape(n, heads, out_ch)
    a_src = jnp.sum(h * att_src, axis=-1)
    a_dst = jnp.sum(h * att_dst, axis=-1)
    alpha = a_src[s] + a_dst[d]
    alpha = jax.nn.leaky_relu(alpha, slope)
    amax = jax.ops.segment_max(alpha, d, num_segments=n)
    amax = jnp.where(jnp.isfinite(amax), amax, 0.0)
    ex = jnp.exp(alpha - amax[d])
    denom = jax.ops.segment_sum(ex, d, num_segments=n)
    coef = ex / (denom[d] + 1e-16)
    msg = h[s] * coef[:, :, None]
    out = jax.ops.segment_sum(msg, d, num_segments=n)
    return out.reshape(n, heads * out_ch) + bias


def kernel(x, edge_index, W1, att_src1, att_dst1, b1, W2, att_src2, att_dst2, b2):
    src = edge_index[0]
    dst = edge_index[1]
    loop = jnp.arange(N, dtype=src.dtype)
    s = jnp.concatenate([src, loop])
    d = jnp.concatenate([dst, loop])
    h = _gat_conv(x, s, d, W1, att_src1, att_dst1, b1, HEADS, HID)
    h = jax.nn.leaky_relu(h, 0.01)
    h = _gat_conv(h, s, d, W2, att_src2, att_dst2, b2, 1, 64)
    return jax.nn.softmax(h, axis=1)


# speedup vs baseline: 27.7904x; 26.9991x over previous
import dataclasses
import functools

import jax
import jax.numpy as jnp
from jax import lax
from jax.experimental import pallas as pl
from jax.experimental.pallas import tpu as pltpu
from jax.experimental.pallas import tpu_sc as plsc

N = 10000
E = 320000
F_IN = 128
HEADS = 8
HID = 16
NUM_CLASSES = 64

N1 = 10240
CHUNK = 96
NTILES = 32
E_TOT = E + N
NCHUNK = -(-E_TOT // (NTILES * CHUNK))
E_PAD = NTILES * NCHUNK * CHUNK
PER_TILE = NCHUNK * CHUNK

_f32 = jnp.float32
_i32 = jnp.int32

_MESH = plsc.VectorSubcoreMesh(core_axis_name="c", subcore_axis_name="s")

_SC_CP = pltpu.CompilerParams()
for _fld, _val in (("needs_layout_passes", False), ("use_tc_tiling_on_sc", False)):
    if _fld in pltpu.CompilerParams.__dataclass_fields__:
        _SC_CP = dataclasses.replace(_SC_CP, **{_fld: _val})


def _sc1_body(h_hbm, a_hbm, b_hbm, si_hbm, di_hbm, z_hbm, out_hbm,
              sidx, didx, av, bv, hv, mv, acc):
    c = lax.axis_index("c")
    s = lax.axis_index("s")
    w = c * 16 + s
    rows = N1 // 16
    pltpu.sync_copy(z_hbm.at[pl.ds(s * rows, rows)],
                    acc.at[pl.ds(s * rows, rows)])
    plsc.subcore_barrier()
    base = w * PER_TILE

    @pl.loop(0, NCHUNK)
    def _(g):
        off = base + g * CHUNK
        pltpu.sync_copy(si_hbm.at[pl.ds(off, CHUNK)], sidx)
        pltpu.sync_copy(di_hbm.at[pl.ds(off, CHUNK)], didx)
        pltpu.sync_copy(a_hbm.at[sidx], av)
        pltpu.sync_copy(b_hbm.at[didx], bv)
        pltpu.sync_copy(h_hbm.at[sidx], hv)

        @pl.loop(0, CHUNK)
        def _(j):
            al = av[j] + bv[j]
            al = jnp.maximum(al, al * 0.2)
            ex = jnp.exp(al)
            mv[j, pl.ds(128, 16)] = ex
            for hh in range(HEADS):
                i0 = jnp.full((16,), j, _i32)
                i1 = jnp.full((16,), 128 + hh, _i32)
                wv = plsc.load_gather(mv, [i0, i1])
                mv[j, pl.ds(16 * hh, 16)] = wv * hv[j, pl.ds(16 * hh, 16)]

        pltpu.sync_copy(mv, acc.at[didx], add=True)

    plsc.subcore_barrier()
    pltpu.sync_copy(acc.at[pl.ds(s * rows, rows)],
                    out_hbm.at[c, pl.ds(s * rows, rows)])


def _sc_layer1(h, a_src, a_dst, si, di, zeros144):
    k = pl.kernel(
        _sc1_body,
        out_type=jax.ShapeDtypeStruct((2, N1, 144), _f32),
        mesh=_MESH,
        compiler_params=_SC_CP,
        scratch_types=[
            pltpu.VMEM((CHUNK,), _i32),
            pltpu.VMEM((CHUNK,), _i32),
            pltpu.VMEM((CHUNK, 16), _f32),
            pltpu.VMEM((CHUNK, 16), _f32),
            pltpu.VMEM((CHUNK, 128), _f32),
            pltpu.VMEM((CHUNK, 144), _f32),
            pltpu.VMEM_SHARED((N1, 144), _f32),
        ],
    )
    return k(h, a_src, a_dst, si, di, zeros144)


def _sc2_body(h_hbm, a_hbm, b_hbm, si_hbm, di_hbm, z_hbm, out_hbm,
              sidx, didx, av, bv, hv, mv, acc):
    c = lax.axis_index("c")
    s = lax.axis_index("s")
    w = c * 16 + s
    rows = N1 // 16
    pltpu.sync_copy(z_hbm.at[pl.ds(s * rows, rows)],
                    acc.at[pl.ds(s * rows, rows)])
    plsc.subcore_barrier()
    base = w * PER_TILE

    @pl.loop(0, NCHUNK)
    def _(g):
        off = base + g * CHUNK
        pltpu.sync_copy(si_hbm.at[pl.ds(off, CHUNK)], sidx)
        pltpu.sync_copy(di_hbm.at[pl.ds(off, CHUNK)], didx)
        pltpu.sync_copy(a_hbm.at[sidx], av)
        pltpu.sync_copy(b_hbm.at[didx], bv)
        pltpu.sync_copy(h_hbm.at[sidx], hv)

        @pl.loop(0, CHUNK)
        def _(j):
            al = av[j] + bv[j]
            al = jnp.maximum(al, al * 0.2)
            ex = jnp.exp(al)
            mv[j, pl.ds(64, 16)] = ex
            for q in range(4):
                mv[j, pl.ds(16 * q, 16)] = ex * hv[j, pl.ds(16 * q, 16)]

        pltpu.sync_copy(mv, acc.at[didx], add=True)

    plsc.subcore_barrier()
    pltpu.sync_copy(acc.at[pl.ds(s * rows, rows)],
                    out_hbm.at[c, pl.ds(s * rows, rows)])


def _sc_layer2(h2, a2s, a2d, si, di, zeros80):
    k = pl.kernel(
        _sc2_body,
        out_type=jax.ShapeDtypeStruct((2, N1, 80), _f32),
        mesh=_MESH,
        compiler_params=_SC_CP,
        scratch_types=[
            pltpu.VMEM((CHUNK,), _i32),
            pltpu.VMEM((CHUNK,), _i32),
            pltpu.VMEM((CHUNK, 16), _f32),
            pltpu.VMEM((CHUNK, 16), _f32),
            pltpu.VMEM((CHUNK, 64), _f32),
            pltpu.VMEM((CHUNK, 80), _f32),
            pltpu.VMEM_SHARED((N1, 80), _f32),
        ],
    )
    return k(h2, a2s, a2d, si, di, zeros80)


_BM = 1024


def _stA_body(x_ref, w_ref, as_ref, ad_ref, h_ref, a_ref, b_ref):
    h = jnp.dot(x_ref[...], w_ref[...], preferred_element_type=_f32)
    h_ref[...] = h
    sa = jnp.dot(h, as_ref[...], preferred_element_type=_f32)
    sb = jnp.dot(h, ad_ref[...], preferred_element_type=_f32)
    neg = jnp.full((_BM, 8), -1e30, _f32)
    a_ref[...] = jnp.concatenate([sa, neg], axis=1)
    b_ref[...] = jnp.concatenate([sb, neg], axis=1)


def _stage_a(x_p, W1, As1, Ad1):
    return pl.pallas_call(
        _stA_body,
        grid=(N1 // _BM,),
        in_specs=[
            pl.BlockSpec((_BM, F_IN), lambda i: (i, 0)),
            pl.BlockSpec((F_IN, 128), lambda i: (0, 0)),
            pl.BlockSpec((128, 8), lambda i: (0, 0)),
            pl.BlockSpec((128, 8), lambda i: (0, 0)),
        ],
        out_specs=[
            pl.BlockSpec((_BM, 128), lambda i: (i, 0)),
            pl.BlockSpec((_BM, 16), lambda i: (i, 0)),
            pl.BlockSpec((_BM, 16), lambda i: (i, 0)),
        ],
        out_shape=[
            jax.ShapeDtypeStruct((N1, 128), _f32),
            jax.ShapeDtypeStruct((N1, 16), _f32),
            jax.ShapeDtypeStruct((N1, 16), _f32),
        ],
    )(x_p, W1, As1, Ad1)


def _stB_body(acc_ref, b1_ref, w2_ref, a2s_ref, a2d_ref, e8_ref,
              h2_ref, sa_ref, sb_ref):
    accs = acc_ref[0] + acc_ref[1]
    msg = accs[:, :128]
    den = accs[:, 128:136]
    den128 = jnp.dot(den, e8_ref[...], preferred_element_type=_f32)
    y = msg / (den128 + 1e-16) + b1_ref[...]
    y = jnp.maximum(y, 0.01 * y)
    h2 = jnp.dot(y, w2_ref[...], preferred_element_type=_f32)
    h2_ref[...] = h2
    sa_ref[...] = jnp.dot(h2, a2s_ref[...], preferred_element_type=_f32)
    sb_ref[...] = jnp.dot(h2, a2d_ref[...], preferred_element_type=_f32)


def _stage_b(acc1, b1, W2, A2s, A2d, E8):
    return pl.pallas_call(
        _stB_body,
        grid=(N1 // _BM,),
        in_specs=[
            pl.BlockSpec((2, _BM, 144), lambda i: (0, i, 0)),
            pl.BlockSpec((1, 128), lambda i: (0, 0)),
            pl.BlockSpec((128, 64), lambda i: (0, 0)),
            pl.BlockSpec((64, 16), lambda i: (0, 0)),
            pl.BlockSpec((64, 16), lambda i: (0, 0)),
            pl.BlockSpec((8, 128), lambda i: (0, 0)),
        ],
        out_specs=[
            pl.BlockSpec((_BM, 64), lambda i: (i, 0)),
            pl.BlockSpec((_BM, 16), lambda i: (i, 0)),
            pl.BlockSpec((_BM, 16), lambda i: (i, 0)),
        ],
        out_shape=[
            jax.ShapeDtypeStruct((N1, 64), _f32),
            jax.ShapeDtypeStruct((N1, 16), _f32),
            jax.ShapeDtypeStruct((N1, 16), _f32),
        ],
    )(acc1, b1, W2, A2s, A2d, E8)


def _stC_body(acc_ref, b2_ref, o_ref):
    accs = acc_ref[0] + acc_ref[1]
    msg = accs[:, :64]
    den = accs[:, 64:65]
    logits = msg / (den + 1e-16) + b2_ref[...]
    m = jnp.max(logits, axis=1, keepdims=True)
    e = jnp.exp(logits - m)
    o_ref[...] = e / jnp.sum(e, axis=1, keepdims=True)


def _stage_c(acc2, b2):
    return pl.pallas_call(
        _stC_body,
        grid=(N1 // _BM,),
        in_specs=[
            pl.BlockSpec((2, _BM, 80), lambda i: (0, i, 0)),
            pl.BlockSpec((1, 64), lambda i: (0, 0)),
        ],
        out_specs=pl.BlockSpec((_BM, 64), lambda i: (i, 0)),
        out_shape=jax.ShapeDtypeStruct((N1, 64), _f32),
    )(acc2, b2)


def kernel(x, edge_index, W1, att_src1, att_dst1, b1, W2, att_src2, att_dst2, b2):
    src = edge_index[0].astype(_i32)
    dst = edge_index[1].astype(_i32)
    loop = jnp.arange(N, dtype=_i32)
    fill = jnp.full((E_PAD - E_TOT,), N, _i32)
    si = jnp.concatenate([src, loop, fill])
    di = jnp.concatenate([dst, loop, fill])

    x_p = jnp.pad(x, ((0, N1 - N), (0, 0)))

    eye8 = jnp.eye(8, dtype=_f32)
    As1 = (att_src1.reshape(8, 16)[:, :, None] * eye8[:, None, :]).reshape(128, 8)
    Ad1 = (att_dst1.reshape(8, 16)[:, :, None] * eye8[:, None, :]).reshape(128, 8)
    E8 = jnp.repeat(eye8, 16, axis=1)
    A2s = jnp.tile(att_src2.reshape(64, 1), (1, 16))
    A2d = jnp.tile(att_dst2.reshape(64, 1), (1, 16))
    zeros144 = jnp.zeros((N1, 144), _f32)
    zeros80 = jnp.zeros((N1, 80), _f32)

    h1, a1s, a1d = _stage_a(x_p, W1, As1, Ad1)
    acc1 = _sc_layer1(h1, a1s, a1d, si, di, zeros144)
    h2, a2s, a2d = _stage_b(acc1, b1.reshape(1, 128), W2, A2s, A2d, E8)
    acc2 = _sc_layer2(h2, a2s, a2d, si, di, zeros80)
    out = _stage_c(acc2, b2.reshape(1, 64))
    return out[:N]


# speedup vs baseline: 52.8097x; 1.9003x over previous
import dataclasses

import jax
import jax.numpy as jnp
from jax import lax
from jax.experimental import pallas as pl
from jax.experimental.pallas import tpu as pltpu
from jax.experimental.pallas import tpu_sc as plsc

N = 10000
E = 320000
F_IN = 128
HEADS = 8
HID = 16
NUM_CLASSES = 64

N1 = 10240
CHUNK = 112
NTILES = 32
E_TOT = E + N
_REAL_CH = -(-E_TOT // (NTILES * CHUNK))
NALL = _REAL_CH + (2 if _REAL_CH % 2 == 0 else 1)
PER_TILE = NALL * CHUNK
E_PAD = NTILES * PER_TILE

_f32 = jnp.float32
_i32 = jnp.int32

_MESH = plsc.VectorSubcoreMesh(core_axis_name="c", subcore_axis_name="s")

_SC_CP = pltpu.CompilerParams()
for _fld, _val in (("needs_layout_passes", False), ("use_tc_tiling_on_sc", False)):
    if _fld in pltpu.CompilerParams.__dataclass_fields__:
        _SC_CP = dataclasses.replace(_SC_CP, **{_fld: _val})


def _make_sc_body(width, fdim):
    heads8 = fdim == 128

    def body(hx_hbm, ad_hbm, si_hbm, di_hbm, z_hbm, out_hbm,
             si0, di0, sdi0, hx0, bv0, si1, di1, sdi1, hx1, bv1, acc,
             is0, gs0, ss0, is1, gs1, ss1):
        c = lax.axis_index("c")
        s = lax.axis_index("s")
        w = c * 16 + s
        rows = N1 // 16
        pltpu.sync_copy(z_hbm.at[pl.ds(s * rows, rows)],
                        acc.at[pl.ds(s * rows, rows)])
        plsc.subcore_barrier()
        base = w * PER_TILE

        bufs = ((si0, di0, sdi0, hx0, bv0, is0, gs0, ss0),
                (si1, di1, sdi1, hx1, bv1, is1, gs1, ss1))

        def idx_start(g, B):
            off = base + g * CHUNK
            pltpu.make_async_copy(si_hbm.at[pl.ds(off, CHUNK)], B[0], B[5]).start()
            pltpu.make_async_copy(di_hbm.at[pl.ds(off, CHUNK)], B[1], B[5]).start()

        def idx_wait(B):
            pltpu.make_async_copy(si_hbm.at[pl.ds(base, CHUNK)], B[0], B[5]).wait()
            pltpu.make_async_copy(di_hbm.at[pl.ds(base, CHUNK)], B[1], B[5]).wait()

        def gat_start(B):
            pltpu.make_async_copy(hx_hbm.at[B[0]], B[3], B[6]).start()
            pltpu.make_async_copy(ad_hbm.at[B[1]], B[4], B[6]).start()

        def gat_wait(B):
            pltpu.make_async_copy(hx_hbm.at[B[0]], B[3], B[6]).wait()
            pltpu.make_async_copy(ad_hbm.at[B[1]], B[4], B[6]).wait()

        def sct_start(B):
            pltpu.make_async_copy(B[3], acc.at[B[2]], B[7]).start(add=True)

        def sct_wait(B):
            pltpu.make_async_copy(B[3], acc.at[B[2]], B[7]).wait()

        def compute(B):
            dib, sdib, hxb, bvb = B[1], B[2], B[3], B[4]

            @pl.loop(0, CHUNK // 16)
            def _(t):
                sdib[pl.ds(t * 16, 16)] = dib[pl.ds(t * 16, 16)]

            @pl.loop(0, CHUNK)
            def _(j):
                al = hxb[j, pl.ds(fdim, 16)] + bvb[j]
                al = jnp.maximum(al, al * 0.2)
                ex = jnp.exp(al)
                hxb[j, pl.ds(fdim, 16)] = ex
                if heads8:
                    for hh in range(8):
                        i0 = jnp.full((16,), j, _i32)
                        i1 = jnp.full((16,), fdim + hh, _i32)
                        wv = plsc.load_gather(hxb, [i0, i1])
                        hxb[j, pl.ds(16 * hh, 16)] = \
                            wv * hxb[j, pl.ds(16 * hh, 16)]
                else:
                    for q in range(4):
                        hxb[j, pl.ds(16 * q, 16)] = \
                            ex * hxb[j, pl.ds(16 * q, 16)]

        idx_start(0, bufs[0])
        idx_start(1, bufs[1])
        idx_wait(bufs[0])
        gat_start(bufs[0])

        @pl.loop(0, NALL, step=2)
        def _(g):
            for half in range(2):
                gg = g + half
                B = bufs[half]
                NB = bufs[1 - half]

                @pl.when(jnp.logical_and(gg >= 1, gg + 1 < NALL))
                def _():
                    sct_wait(NB)

                @pl.when(gg + 1 < NALL)
                def _():
                    idx_wait(NB)
                    gat_start(NB)

                gat_wait(B)
                compute(B)
                sct_start(B)

                @pl.when(gg + 2 < NALL)
                def _():
                    idx_start(gg + 2, B)

        sct_wait(bufs[0])
        sct_wait(bufs[1])
        plsc.subcore_barrier()
        pltpu.sync_copy(acc.at[pl.ds(s * rows, rows)],
                        out_hbm.at[c, pl.ds(s * rows, rows)])

    return body


def _sc_layer(hx, ad, si, di, zeros, width, fdim):
    k = pl.kernel(
        _make_sc_body(width, fdim),
        out_type=jax.ShapeDtypeStruct((2, N1, width), _f32),
        mesh=_MESH,
        compiler_params=_SC_CP,
        scratch_types=[
            pltpu.VMEM((CHUNK,), _i32),
            pltpu.VMEM((CHUNK,), _i32),
            pltpu.VMEM((CHUNK,), _i32),
            pltpu.VMEM((CHUNK, width), _f32),
            pltpu.VMEM((CHUNK, 16), _f32),
            pltpu.VMEM((CHUNK,), _i32),
            pltpu.VMEM((CHUNK,), _i32),
            pltpu.VMEM((CHUNK,), _i32),
            pltpu.VMEM((CHUNK, width), _f32),
            pltpu.VMEM((CHUNK, 16), _f32),
            pltpu.VMEM_SHARED((N1, width), _f32),
            pltpu.SemaphoreType.DMA,
            pltpu.SemaphoreType.DMA,
            pltpu.SemaphoreType.DMA,
            pltpu.SemaphoreType.DMA,
            pltpu.SemaphoreType.DMA,
            pltpu.SemaphoreType.DMA,
        ],
    )
    return k(hx, ad, si, di, zeros)


_BM = 1024


def _stA_body(x_ref, w_ref, as_ref, ad_ref, hx_ref, b_ref):
    h = jnp.dot(x_ref[...], w_ref[...], preferred_element_type=_f32)
    sa = jnp.dot(h, as_ref[...], preferred_element_type=_f32)
    sb = jnp.dot(h, ad_ref[...], preferred_element_type=_f32)
    neg = jnp.full((_BM, 8), -1e30, _f32)
    hx_ref[...] = jnp.concatenate([h, sa, neg], axis=1)
    b_ref[...] = jnp.concatenate([sb, neg], axis=1)


def _stage_a(x_p, W1, As1, Ad1):
    return pl.pallas_call(
        _stA_body,
        grid=(N1 // _BM,),
        in_specs=[
            pl.BlockSpec((_BM, F_IN), lambda i: (i, 0)),
            pl.BlockSpec((F_IN, 128), lambda i: (0, 0)),
            pl.BlockSpec((128, 8), lambda i: (0, 0)),
            pl.BlockSpec((128, 8), lambda i: (0, 0)),
        ],
        out_specs=[
            pl.BlockSpec((_BM, 144), lambda i: (i, 0)),
            pl.BlockSpec((_BM, 16), lambda i: (i, 0)),
        ],
        out_shape=[
            jax.ShapeDtypeStruct((N1, 144), _f32),
            jax.ShapeDtypeStruct((N1, 16), _f32),
        ],
    )(x_p, W1, As1, Ad1)


def _stB_body(acc_ref, b1_ref, w2_ref, a2s_ref, a2d_ref, e8_ref,
              hx_ref, b_ref):
    accs = acc_ref[0] + acc_ref[1]
    msg = accs[:, :128]
    den = accs[:, 128:136]
    den128 = jnp.dot(den, e8_ref[...], preferred_element_type=_f32)
    y = msg / (den128 + 1e-16) + b1_ref[...]
    y = jnp.maximum(y, 0.01 * y)
    h2 = jnp.dot(y, w2_ref[...], preferred_element_type=_f32)
    sa = jnp.dot(h2, a2s_ref[...], preferred_element_type=_f32)
    hx_ref[...] = jnp.concatenate([h2, sa], axis=1)
    b_ref[...] = jnp.dot(h2, a2d_ref[...], preferred_element_type=_f32)


def _stage_b(acc1, b1, W2, A2s, A2d, E8):
    return pl.pallas_call(
        _stB_body,
        grid=(N1 // _BM,),
        in_specs=[
            pl.BlockSpec((2, _BM, 144), lambda i: (0, i, 0)),
            pl.BlockSpec((1, 128), lambda i: (0, 0)),
            pl.BlockSpec((128, 64), lambda i: (0, 0)),
            pl.BlockSpec((64, 16), lambda i: (0, 0)),
            pl.BlockSpec((64, 16), lambda i: (0, 0)),
            pl.BlockSpec((8, 128), lambda i: (0, 0)),
        ],
        out_specs=[
            pl.BlockSpec((_BM, 80), lambda i: (i, 0)),
            pl.BlockSpec((_BM, 16), lambda i: (i, 0)),
        ],
        out_shape=[
            jax.ShapeDtypeStruct((N1, 80), _f32),
            jax.ShapeDtypeStruct((N1, 16), _f32),
        ],
    )(acc1, b1, W2, A2s, A2d, E8)


def _stC_body(acc_ref, b2_ref, o_ref):
    accs = acc_ref[0] + acc_ref[1]
    msg = accs[:, :64]
    den = accs[:, 64:65]
    logits = msg / (den + 1e-16) + b2_ref[...]
    m = jnp.max(logits, axis=1, keepdims=True)
    e = jnp.exp(logits - m)
    o_ref[...] = e / jnp.sum(e, axis=1, keepdims=True)


def _stage_c(acc2, b2):
    return pl.pallas_call(
        _stC_body,
        grid=(N1 // _BM,),
        in_specs=[
            pl.BlockSpec((2, _BM, 80), lambda i: (0, i, 0)),
            pl.BlockSpec((1, 64), lambda i: (0, 0)),
        ],
        out_specs=pl.BlockSpec((_BM, 64), lambda i: (i, 0)),
        out_shape=jax.ShapeDtypeStruct((N1, 64), _f32),
    )(acc2, b2)


def kernel(x, edge_index, W1, att_src1, att_dst1, b1, W2, att_src2, att_dst2, b2):
    src = edge_index[0].astype(_i32)
    dst = edge_index[1].astype(_i32)
    loop = jnp.arange(N, dtype=_i32)
    fill = N + (jnp.arange(E_PAD - E_TOT, dtype=_i32) % (N1 - N))
    si = jnp.concatenate([src, loop, fill])
    di = jnp.concatenate([dst, loop, fill])

    x_p = jnp.pad(x, ((0, N1 - N), (0, 0)))

    eye8 = jnp.eye(8, dtype=_f32)
    As1 = (att_src1.reshape(8, 16)[:, :, None] * eye8[:, None, :]).reshape(128, 8)
    Ad1 = (att_dst1.reshape(8, 16)[:, :, None] * eye8[:, None, :]).reshape(128, 8)
    E8 = jnp.repeat(eye8, 16, axis=1)
    A2s = jnp.tile(att_src2.reshape(64, 1), (1, 16))
    A2d = jnp.tile(att_dst2.reshape(64, 1), (1, 16))
    zeros144 = jnp.zeros((N1, 144), _f32)
    zeros80 = jnp.zeros((N1, 80), _f32)

    hx1, ad1 = _stage_a(x_p, W1, As1, Ad1)
    acc1 = _sc_layer(hx1, ad1, si, di, zeros144, 144, 128)
    hx2, ad2 = _stage_b(acc1, b1.reshape(1, 128), W2, A2s, A2d, E8)
    acc2 = _sc_layer(hx2, ad2, si, di, zeros80, 80, 64)
    out = _stage_c(acc2, b2.reshape(1, 64))
    return out[:N]


# speedup vs baseline: 125.9384x; 2.3848x over previous
import dataclasses

import jax
import jax.numpy as jnp
from jax import lax
from jax.experimental import pallas as pl
from jax.experimental.pallas import tpu as pltpu
from jax.experimental.pallas import tpu_sc as plsc

N = 10000
E = 320000
F_IN = 128
HEADS = 8
HID = 16
NUM_CLASSES = 64

N1 = 10240
CHUNK = 112
NTILES = 32
E_TOT = E + N
_REAL_CH = -(-E_TOT // (NTILES * CHUNK))
NALL = _REAL_CH + (2 if _REAL_CH % 2 == 0 else 1)
PER_TILE = NALL * CHUNK
E_PAD = NTILES * PER_TILE

_f32 = jnp.float32
_i32 = jnp.int32

_MESH = plsc.VectorSubcoreMesh(core_axis_name="c", subcore_axis_name="s")

_GDN = lax.GatherDimensionNumbers(
    offset_dims=(), collapsed_slice_dims=(0,), start_index_map=(0,))

_SC_CP = pltpu.CompilerParams()
for _fld, _val in (("needs_layout_passes", False), ("use_tc_tiling_on_sc", False)):
    if _fld in pltpu.CompilerParams.__dataclass_fields__:
        _SC_CP = dataclasses.replace(_SC_CP, **{_fld: _val})


def _make_sc_body(width, fdim):
    heads8 = fdim == 128

    def body(hx_hbm, ad_hbm, si_hbm, di_hbm, z_hbm, out_hbm,
             si0, di0, sdi0, hx0, bv0, si1, di1, sdi1, hx1, bv1, acc,
             is0, gs0, ss0, is1, gs1, ss1):
        c = lax.axis_index("c")
        s = lax.axis_index("s")
        w = c * 16 + s
        rows = N1 // 16
        pltpu.sync_copy(z_hbm.at[pl.ds(s * rows, rows)],
                        acc.at[pl.ds(s * rows, rows)])
        plsc.subcore_barrier()
        base = w * PER_TILE

        bufs = ((si0, di0, sdi0, hx0, bv0, is0, gs0, ss0),
                (si1, di1, sdi1, hx1, bv1, is1, gs1, ss1))

        def idx_start(g, B):
            off = base + g * CHUNK
            pltpu.make_async_copy(si_hbm.at[pl.ds(off, CHUNK)], B[0], B[5]).start()
            pltpu.make_async_copy(di_hbm.at[pl.ds(off, CHUNK)], B[1], B[5]).start()

        def idx_wait(B):
            pltpu.make_async_copy(si_hbm.at[pl.ds(base, CHUNK)], B[0], B[5]).wait()
            pltpu.make_async_copy(di_hbm.at[pl.ds(base, CHUNK)], B[1], B[5]).wait()

        def gat_start(B):
            pltpu.make_async_copy(hx_hbm.at[B[0]], B[3], B[6]).start()
            pltpu.make_async_copy(ad_hbm.at[B[1]], B[4], B[6]).start()

        def gat_wait(B):
            pltpu.make_async_copy(hx_hbm.at[B[0]], B[3], B[6]).wait()
            pltpu.make_async_copy(ad_hbm.at[B[1]], B[4], B[6]).wait()

        def sct_start(B):
            pltpu.make_async_copy(B[3], acc.at[B[2]], B[7]).start(add=True)

        def sct_wait(B):
            pltpu.make_async_copy(B[3], acc.at[B[2]], B[7]).wait()

        def compute(B):
            dib, sdib, hxb, bvb = B[1], B[2], B[3], B[4]

            @pl.loop(0, CHUNK // 16)
            def _(t):
                sdib[pl.ds(t * 16, 16)] = dib[pl.ds(t * 16, 16)]

            @plsc.parallel_loop(0, CHUNK, unroll=2)
            def _(j):
                al = hxb[j, pl.ds(fdim, 16)] + bvb[j]
                al = jnp.maximum(al, al * 0.2)
                ex = jnp.exp(al)
                hxb[j, pl.ds(fdim, 16)] = ex
                if heads8:
                    for hh in range(8):
                        lane = jnp.full((16, 1), hh, _i32)
                        wv = lax.gather(
                            ex, lane, _GDN, slice_sizes=(1,),
                            mode=lax.GatherScatterMode.PROMISE_IN_BOUNDS)
                        hxb[j, pl.ds(16 * hh, 16)] = \
                            wv * hxb[j, pl.ds(16 * hh, 16)]
                else:
                    for q in range(4):
                        hxb[j, pl.ds(16 * q, 16)] = \
                            ex * hxb[j, pl.ds(16 * q, 16)]

        idx_start(0, bufs[0])
        idx_start(1, bufs[1])
        idx_wait(bufs[0])
        gat_start(bufs[0])

        @pl.loop(0, NALL, step=2)
        def _(g):
            for half in range(2):
                gg = g + half
                B = bufs[half]
                NB = bufs[1 - half]

                @pl.when(jnp.logical_and(gg >= 1, gg + 1 < NALL))
                def _():
                    sct_wait(NB)

                @pl.when(gg + 1 < NALL)
                def _():
                    idx_wait(NB)
                    gat_start(NB)

                gat_wait(B)
                compute(B)
                sct_start(B)

                @pl.when(gg + 2 < NALL)
                def _():
                    idx_start(gg + 2, B)

        sct_wait(bufs[0])
        sct_wait(bufs[1])
        plsc.subcore_barrier()
        pltpu.sync_copy(acc.at[pl.ds(s * rows, rows)],
                        out_hbm.at[c, pl.ds(s * rows, rows)])

    return body


def _sc_layer(hx, ad, si, di, zeros, width, fdim):
    k = pl.kernel(
        _make_sc_body(width, fdim),
        out_type=jax.ShapeDtypeStruct((2, N1, width), _f32),
        mesh=_MESH,
        compiler_params=_SC_CP,
        scratch_types=[
            pltpu.VMEM((CHUNK,), _i32),
            pltpu.VMEM((CHUNK,), _i32),
            pltpu.VMEM((CHUNK,), _i32),
            pltpu.VMEM((CHUNK, width), _f32),
            pltpu.VMEM((CHUNK, 16), _f32),
            pltpu.VMEM((CHUNK,), _i32),
            pltpu.VMEM((CHUNK,), _i32),
            pltpu.VMEM((CHUNK,), _i32),
            pltpu.VMEM((CHUNK, width), _f32),
            pltpu.VMEM((CHUNK, 16), _f32),
            pltpu.VMEM_SHARED((N1, width), _f32),
            pltpu.SemaphoreType.DMA,
            pltpu.SemaphoreType.DMA,
            pltpu.SemaphoreType.DMA,
            pltpu.SemaphoreType.DMA,
            pltpu.SemaphoreType.DMA,
            pltpu.SemaphoreType.DMA,
        ],
    )
    return k(hx, ad, si, di, zeros)


_BM = 1024


def _stA_body(x_ref, w_ref, as_ref, ad_ref, hx_ref, b_ref):
    h = jnp.dot(x_ref[...], w_ref[...], preferred_element_type=_f32)
    sa = jnp.dot(h, as_ref[...], preferred_element_type=_f32)
    sb = jnp.dot(h, ad_ref[...], preferred_element_type=_f32)
    neg = jnp.full((_BM, 8), -1e30, _f32)
    hx_ref[...] = jnp.concatenate([h, sa, neg], axis=1)
    b_ref[...] = jnp.concatenate([sb, neg], axis=1)


def _stage_a(x_p, W1, As1, Ad1):
    return pl.pallas_call(
        _stA_body,
        grid=(N1 // _BM,),
        in_specs=[
            pl.BlockSpec((_BM, F_IN), lambda i: (i, 0)),
            pl.BlockSpec((F_IN, 128), lambda i: (0, 0)),
            pl.BlockSpec((128, 8), lambda i: (0, 0)),
            pl.BlockSpec((128, 8), lambda i: (0, 0)),
        ],
        out_specs=[
            pl.BlockSpec((_BM, 144), lambda i: (i, 0)),
            pl.BlockSpec((_BM, 16), lambda i: (i, 0)),
        ],
        out_shape=[
            jax.ShapeDtypeStruct((N1, 144), _f32),
            jax.ShapeDtypeStruct((N1, 16), _f32),
        ],
    )(x_p, W1, As1, Ad1)


def _stB_body(acc_ref, b1_ref, w2_ref, a2s_ref, a2d_ref, e8_ref,
              hx_ref, b_ref):
    accs = acc_ref[0] + acc_ref[1]
    msg = accs[:, :128]
    den = accs[:, 128:136]
    den128 = jnp.dot(den, e8_ref[...], preferred_element_type=_f32)
    y = msg / (den128 + 1e-16) + b1_ref[...]
    y = jnp.maximum(y, 0.01 * y)
    h2 = jnp.dot(y, w2_ref[...], preferred_element_type=_f32)
    sa = jnp.dot(h2, a2s_ref[...], preferred_element_type=_f32)
    hx_ref[...] = jnp.concatenate([h2, sa], axis=1)
    b_ref[...] = jnp.dot(h2, a2d_ref[...], preferred_element_type=_f32)


def _stage_b(acc1, b1, W2, A2s, A2d, E8):
    return pl.pallas_call(
        _stB_body,
        grid=(N1 // _BM,),
        in_specs=[
            pl.BlockSpec((2, _BM, 144), lambda i: (0, i, 0)),
            pl.BlockSpec((1, 128), lambda i: (0, 0)),
            pl.BlockSpec((128, 64), lambda i: (0, 0)),
            pl.BlockSpec((64, 16), lambda i: (0, 0)),
            pl.BlockSpec((64, 16), lambda i: (0, 0)),
            pl.BlockSpec((8, 128), lambda i: (0, 0)),
        ],
        out_specs=[
            pl.BlockSpec((_BM, 80), lambda i: (i, 0)),
            pl.BlockSpec((_BM, 16), lambda i: (i, 0)),
        ],
        out_shape=[
            jax.ShapeDtypeStruct((N1, 80), _f32),
            jax.ShapeDtypeStruct((N1, 16), _f32),
        ],
    )(acc1, b1, W2, A2s, A2d, E8)


def _stC_body(acc_ref, b2_ref, o_ref):
    accs = acc_ref[0] + acc_ref[1]
    msg = accs[:, :64]
    den = accs[:, 64:65]
    logits = msg / (den + 1e-16) + b2_ref[...]
    m = jnp.max(logits, axis=1, keepdims=True)
    e = jnp.exp(logits - m)
    o_ref[...] = e / jnp.sum(e, axis=1, keepdims=True)


def _stage_c(acc2, b2):
    return pl.pallas_call(
        _stC_body,
        grid=(N1 // _BM,),
        in_specs=[
            pl.BlockSpec((2, _BM, 80), lambda i: (0, i, 0)),
            pl.BlockSpec((1, 64), lambda i: (0, 0)),
        ],
        out_specs=pl.BlockSpec((_BM, 64), lambda i: (i, 0)),
        out_shape=jax.ShapeDtypeStruct((N1, 64), _f32),
    )(acc2, b2)


def kernel(x, edge_index, W1, att_src1, att_dst1, b1, W2, att_src2, att_dst2, b2):
    src = edge_index[0].astype(_i32)
    dst = edge_index[1].astype(_i32)
    loop = jnp.arange(N, dtype=_i32)
    fill = N + (jnp.arange(E_PAD - E_TOT, dtype=_i32) % (N1 - N))
    si = jnp.concatenate([src, loop, fill])
    di = jnp.concatenate([dst, loop, fill])

    x_p = jnp.pad(x, ((0, N1 - N), (0, 0)))

    eye8 = jnp.eye(8, dtype=_f32)
    As1 = (att_src1.reshape(8, 16)[:, :, None] * eye8[:, None, :]).reshape(128, 8)
    Ad1 = (att_dst1.reshape(8, 16)[:, :, None] * eye8[:, None, :]).reshape(128, 8)
    E8 = jnp.repeat(eye8, 16, axis=1)
    A2s = jnp.tile(att_src2.reshape(64, 1), (1, 16))
    A2d = jnp.tile(att_dst2.reshape(64, 1), (1, 16))
    zeros144 = jnp.zeros((N1, 144), _f32)
    zeros80 = jnp.zeros((N1, 80), _f32)

    hx1, ad1 = _stage_a(x_p, W1, As1, Ad1)
    acc1 = _sc_layer(hx1, ad1, si, di, zeros144, 144, 128)
    hx2, ad2 = _stage_b(acc1, b1.reshape(1, 128), W2, A2s, A2d, E8)
    acc2 = _sc_layer(hx2, ad2, si, di, zeros80, 80, 64)
    out = _stage_c(acc2, b2.reshape(1, 64))
    return out[:N]
